# Initial kernel scaffold; baseline (speedup 1.0000x reference)
import jax
import jax.numpy as jnp
from jax.experimental import pallas as pl


def kernel(x, edge_index, edge_type, batch, type_, emb0, emb1, emb2, emb3, emb4, emb5, W1, root1, b1, W2, root2, b2, gate_l1_w, gate_l1_b, bn_gamma, bn_beta, gate_l2_w, gate_l2_b, graph_w, graph_b):
    raise NotImplementedError("write your pallas kernel here")


# speedup vs baseline: 2.9578x; 2.9578x over previous
import functools

import jax
import jax.numpy as jnp
from jax import lax
from jax.experimental import pallas as pl
from jax.experimental.pallas import tpu as pltpu
from jax.experimental.pallas import tpu_sc as plsc

N = 10000
E = 320000
R = 4
IN = 128
LAYER = 512
OUT = 256
NG = 16
DIN = 6 * IN
DCAT = R * LAYER
NR = N * R
NRP = 40960
NB = 1000
GRID = N // NB
EPT = E // 32
CHK = 128
EP = 327680
NCH = EP // 16 // CHK
NROW = N // 16

_f32 = jnp.float32
_i32 = jnp.int32


def _prep_body(b_ref, w1_ref, r1_ref, wx_ref, rx_ref):
    b = b_ref[...]
    wx_ref[...] = jnp.dot(b, w1_ref[...], preferred_element_type=_f32)
    rx_ref[...] = jnp.dot(b, r1_ref[...], preferred_element_type=_f32)


def _tc_prep(bmat, w1cat, root1):
    return pl.pallas_call(
        _prep_body,
        out_shape=(jax.ShapeDtypeStruct((128, DCAT), _f32),
                   jax.ShapeDtypeStruct((128, LAYER), _f32)),
    )(bmat, w1cat, root1)


def _l1_body(xo_ref, wx_ref, rx_ref, b1_ref, hr_ref, pre_ref):
    xo = xo_ref[...]
    col = lax.broadcasted_iota(_i32, (NB, 128), 1)
    oh = jnp.zeros((NB, 128), _f32)
    for i in range(8):
        oh = oh + (col == xo[:, i:i + 1]).astype(_f32)
    hr_ref[...] = jnp.dot(oh, wx_ref[...], preferred_element_type=_f32)
    pre_ref[...] = (jnp.dot(oh, rx_ref[...], preferred_element_type=_f32)
                    + b1_ref[...])


def _tc_layer1(xoff, wx, rx, b1):
    return pl.pallas_call(
        _l1_body,
        grid=(GRID,),
        in_specs=[
            pl.BlockSpec((NB, 8), lambda i: (i, 0)),
            pl.BlockSpec((128, DCAT), lambda i: (0, 0)),
            pl.BlockSpec((128, LAYER), lambda i: (0, 0)),
            pl.BlockSpec((1, LAYER), lambda i: (0, 0)),
        ],
        out_specs=(pl.BlockSpec((NB, DCAT), lambda i: (i, 0)),
                   pl.BlockSpec((NB, LAYER), lambda i: (i, 0))),
        out_shape=(jax.ShapeDtypeStruct((N, DCAT), _f32),
                   jax.ShapeDtypeStruct((N, LAYER), _f32)),
    )(xoff, wx, rx, b1)


def _l2_body(pre_ref, acc_ref, w2_ref, r2_ref, b2_ref, hr_ref, pre2_ref):
    acc = acc_ref[...]
    agg = jnp.concatenate([acc[k] for k in range(4)], axis=1)
    h = jax.nn.sigmoid(pre_ref[...] + agg)
    hr_ref[...] = jnp.dot(h, w2_ref[...], preferred_element_type=_f32)
    pre2_ref[...] = (jnp.dot(h, r2_ref[...], preferred_element_type=_f32)
                     + b2_ref[...])


def _tc_layer2(pre1, acc1, w2cat, root2, b2):
    return pl.pallas_call(
        _l2_body,
        grid=(GRID,),
        in_specs=[
            pl.BlockSpec((NB, LAYER), lambda i: (i, 0)),
            pl.BlockSpec((4, NB, 128), lambda i: (0, i, 0)),
            pl.BlockSpec((LAYER, DCAT), lambda i: (0, 0)),
            pl.BlockSpec((LAYER, LAYER), lambda i: (0, 0)),
            pl.BlockSpec((1, LAYER), lambda i: (0, 0)),
        ],
        out_specs=(pl.BlockSpec((NB, DCAT), lambda i: (i, 0)),
                   pl.BlockSpec((NB, LAYER), lambda i: (i, 0))),
        out_shape=(jax.ShapeDtypeStruct((N, DCAT), _f32),
                   jax.ShapeDtypeStruct((N, LAYER), _f32)),
    )(pre1, acc1, w2cat, root2, b2)


def _post_body(pre_ref, acc_ref, gw_ref, gb_ref, mu_ref, g_ref, sums_ref,
               s_acc):
    i = pl.program_id(0)
    acc = acc_ref[...]
    agg = jnp.concatenate([acc[k] for k in range(4)], axis=1)
    h = jax.nn.sigmoid(pre_ref[...] + agg)
    mu = h[:, :OUT]
    g = jnp.dot(mu, gw_ref[...], preferred_element_type=_f32) + gb_ref[...]
    mu_ref[...] = mu
    g_ref[...] = g

    @pl.when(i == 0)
    def _():
        s_acc[...] = jnp.zeros_like(s_acc)

    part = jnp.concatenate([jnp.sum(g, axis=0, keepdims=True),
                            jnp.sum(g * g, axis=0, keepdims=True)], axis=0)
    s_acc[...] += part

    @pl.when(i == GRID - 1)
    def _():
        sums_ref[...] = s_acc[...]


def _tc_post(pre2, acc2, gate_l1_w, gate_l1_b):
    return pl.pallas_call(
        _post_body,
        grid=(GRID,),
        in_specs=[
            pl.BlockSpec((NB, LAYER), lambda i: (i, 0)),
            pl.BlockSpec((4, NB, 128), lambda i: (0, i, 0)),
            pl.BlockSpec((OUT, OUT), lambda i: (0, 0)),
            pl.BlockSpec((1, OUT), lambda i: (0, 0)),
        ],
        out_specs=(pl.BlockSpec((NB, OUT), lambda i: (i, 0)),
                   pl.BlockSpec((NB, OUT), lambda i: (i, 0)),
                   pl.BlockSpec((2, OUT), lambda i: (0, 0))),
        out_shape=(jax.ShapeDtypeStruct((N, OUT), _f32),
                   jax.ShapeDtypeStruct((N, OUT), _f32),
                   jax.ShapeDtypeStruct((2, OUT), _f32)),
        scratch_shapes=[pltpu.VMEM((2, OUT), _f32)],
    )(pre2, acc2, gate_l1_w, gate_l1_b)


def _gate_body(g_ref, b_ref, sums_ref, gam_ref, bet_ref, w2_ref, gate_ref,
               gmax_ref, m_acc):
    i = pl.program_id(0)
    sums = sums_ref[...]
    mean = sums[0:1, :] / N
    var = sums[1:2, :] / N - mean * mean
    gn = (g_ref[...] - mean) * lax.rsqrt(var + 1e-5) * gam_ref[...] + bet_ref[...]
    gn = jnp.maximum(gn, 0.0)
    gate = jnp.sum(gn * w2_ref[...], axis=1, keepdims=True)
    gate_ref[...] = gate
    seg = lax.broadcasted_iota(_i32, (NB, NG), 1).astype(_f32)
    m = (b_ref[...] == seg)
    masked = jnp.where(m, jnp.broadcast_to(gate, (NB, NG)), -1e30)

    @pl.when(i == 0)
    def _():
        m_acc[...] = jnp.full_like(m_acc, -1e30)

    m_acc[...] = jnp.maximum(m_acc[...], jnp.max(masked, axis=0,
                                                 keepdims=True))

    @pl.when(i == GRID - 1)
    def _():
        gmax_ref[...] = m_acc[...]


def _tc_gate(g, bcol, sums, bn_gamma, bn_beta, w2row):
    return pl.pallas_call(
        _gate_body,
        grid=(GRID,),
        in_specs=[
            pl.BlockSpec((NB, OUT), lambda i: (i, 0)),
            pl.BlockSpec((NB, 1), lambda i: (i, 0)),
            pl.BlockSpec((2, OUT), lambda i: (0, 0)),
            pl.BlockSpec((1, OUT), lambda i: (0, 0)),
            pl.BlockSpec((1, OUT), lambda i: (0, 0)),
            pl.BlockSpec((1, OUT), lambda i: (0, 0)),
        ],
        out_specs=(pl.BlockSpec((NB, 1), lambda i: (i, 0)),
                   pl.BlockSpec((1, NG), lambda i: (0, 0))),
        out_shape=(jax.ShapeDtypeStruct((N, 1), _f32),
                   jax.ShapeDtypeStruct((1, NG), _f32)),
        scratch_shapes=[pltpu.VMEM((1, NG), _f32)],
    )(g, bcol, sums, bn_gamma, bn_beta, w2row)


def _final_body(gate_ref, b_ref, gmax_ref, mu_ref, gw_ref, gb_ref, out_ref,
                u_acc, d_acc):
    i = pl.program_id(0)

    @pl.when(i == 0)
    def _():
        u_acc[...] = jnp.zeros_like(u_acc)
        d_acc[...] = jnp.zeros_like(d_acc)

    seg = lax.broadcasted_iota(_i32, (NB, NG), 1).astype(_f32)
    m = (b_ref[...] == seg).astype(_f32)
    gmb = jnp.sum(m * gmax_ref[...], axis=1, keepdims=True)
    ex = jnp.exp(gate_ref[...] - gmb)
    dn = (((0,), (0,)), ((), ()))
    d_acc[...] += lax.dot_general(m, ex, dn, preferred_element_type=_f32)
    u_acc[...] += lax.dot_general(m, ex * mu_ref[...], dn,
                                  preferred_element_type=_f32)

    @pl.when(i == GRID - 1)
    def _():
        pooled = u_acc[...] / jnp.maximum(d_acc[...], 1e-30)
        val = jnp.dot(pooled, gw_ref[...], preferred_element_type=_f32)
        out_ref[...] = jax.nn.sigmoid(val + gb_ref[...])


def _tc_final(gate, bcol, gmax, mu, graph_w, graph_b):
    return pl.pallas_call(
        _final_body,
        grid=(GRID,),
        in_specs=[
            pl.BlockSpec((NB, 1), lambda i: (i, 0)),
            pl.BlockSpec((NB, 1), lambda i: (i, 0)),
            pl.BlockSpec((1, NG), lambda i: (0, 0)),
            pl.BlockSpec((NB, OUT), lambda i: (i, 0)),
            pl.BlockSpec((OUT, 1), lambda i: (0, 0)),
            pl.BlockSpec((1, 1), lambda i: (0, 0)),
        ],
        out_specs=pl.BlockSpec((NG, 1), lambda i: (0, 0)),
        out_shape=jax.ShapeDtypeStruct((NG, 1), _f32),
        scratch_shapes=[pltpu.VMEM((NG, OUT), _f32),
                        pltpu.VMEM((NG, 1), _f32)],
    )(gate, bcol, gmax, mu, graph_w, graph_b)


_MESH = plsc.VectorSubcoreMesh(core_axis_name="c", subcore_axis_name="s")


def _sc_counts(dst, et):
    @functools.partial(
        pl.kernel,
        out_type=jax.ShapeDtypeStruct((32, NRP), _f32),
        mesh=_MESH,
        compiler_params=pltpu.CompilerParams(needs_layout_passes=False),
        scratch_types=[pltpu.VMEM((EPT,), _i32),
                       pltpu.VMEM((EPT,), _i32),
                       pltpu.VMEM((NRP,), _f32)],
    )
    def k(dst_h, et_h, out_h, dv, tv, cnt):
        c = lax.axis_index("c")
        s = lax.axis_index("s")
        wid = s * 2 + c
        base = wid * EPT
        pltpu.sync_copy(dst_h.at[pl.ds(base, EPT)], dv)
        pltpu.sync_copy(et_h.at[pl.ds(base, EPT)], tv)
        zv = jnp.zeros((16,), _f32)

        def zbody(i, _):
            cnt[pl.ds(i * 16, 16)] = zv
            return ()

        lax.fori_loop(0, NRP // 16, zbody, (), unroll=8)
        ones = jnp.ones((16,), _f32)

        def body(i, _):
            d = dv[pl.ds(i * 16, 16)]
            t = tv[pl.ds(i * 16, 16)]
            plsc.addupdate_scatter(cnt, [d * R + t], ones)
            return ()

        lax.fori_loop(0, EPT // 16, body, (), unroll=4)
        pltpu.sync_copy(cnt, out_h.at[wid])

    return k(dst, et)


def _sc_edgeprep(parts, dst, et, src):
    @functools.partial(
        pl.kernel,
        out_type=(jax.ShapeDtypeStruct((E,), _i32),
                  jax.ShapeDtypeStruct((E,), _f32),
                  jax.ShapeDtypeStruct((E,), _f32),
                  jax.ShapeDtypeStruct((E,), _i32)),
        mesh=_MESH,
        compiler_params=pltpu.CompilerParams(needs_layout_passes=False),
        scratch_types=[pltpu.VMEM((NRP,), _f32),
                       pltpu.VMEM((1280,), _f32),
                       pltpu.VMEM((1280,), _f32),
                       pltpu.VMEM((EPT,), _i32),
                       pltpu.VMEM((EPT,), _i32),
                       pltpu.VMEM((EPT,), _i32),
                       pltpu.VMEM((EPT,), _i32),
                       pltpu.VMEM((EPT,), _f32),
                       pltpu.VMEM((EPT,), _f32),
                       pltpu.VMEM((EPT,), _i32),
                       pltpu.VMEM_SHARED((NRP,), _f32)],
    )
    def k(parts_h, dst_h, et_h, src_h, base_h, s0_h, s1_h, dl_h,
          recip_l, cbuf, pbuf, dv, tv, sv, bb, s0b, s1b, dlb, recip_sh):
        c = lax.axis_index("c")
        s = lax.axis_index("s")
        wid = s * 2 + c
        zv = jnp.zeros((16,), _f32)
        for rep in range(2):
            cid = rep * 16 + s
            off = cid * 1280

            def zbody(i, _):
                cbuf[pl.ds(i * 16, 16)] = zv
                return ()

            lax.fori_loop(0, 80, zbody, (), unroll=8)

            def pad(p, _):
                pltpu.sync_copy(parts_h.at[p, pl.ds(off, 1280)], pbuf)

                def abody(i, _):
                    sl = pl.ds(i * 16, 16)
                    cbuf[sl] = cbuf[sl] + pbuf[sl]
                    return ()

                lax.fori_loop(0, 80, abody, (), unroll=8)
                return ()

            lax.fori_loop(0, 32, pad, ())

            def rbody(i, _):
                sl = pl.ds(i * 16, 16)
                cbuf[sl] = 1.0 / jnp.maximum(cbuf[sl], 1.0)
                return ()

            lax.fori_loop(0, 80, rbody, (), unroll=8)
            pltpu.sync_copy(cbuf, recip_sh.at[pl.ds(off, 1280)])
        plsc.subcore_barrier()
        pltpu.sync_copy(recip_sh, recip_l)
        base = wid * EPT
        pltpu.sync_copy(dst_h.at[pl.ds(base, EPT)], dv)
        pltpu.sync_copy(et_h.at[pl.ds(base, EPT)], tv)
        pltpu.sync_copy(src_h.at[pl.ds(base, EPT)], sv)
        half = N // 2

        def body(i, _):
            sl = pl.ds(i * 16, 16)
            d = dv[sl]
            t = tv[sl]
            r = plsc.load_gather(recip_l, [d * R + t])
            lo = d < half
            s0 = jnp.where(lo, r, 0.0)
            s0b[sl] = s0
            s1b[sl] = r - s0
            dlb[sl] = jnp.where(lo, d, d - half)
            bb[sl] = sv[sl] * R + t
            return ()

        lax.fori_loop(0, EPT // 16, body, (), unroll=2)
        pltpu.sync_copy(bb, base_h.at[pl.ds(base, EPT)])
        pltpu.sync_copy(s0b, s0_h.at[pl.ds(base, EPT)])
        pltpu.sync_copy(s1b, s1_h.at[pl.ds(base, EPT)])
        pltpu.sync_copy(dlb, dl_h.at[pl.ds(base, EPT)])

    return k(parts, dst, et, src)


def _sc_aggregate(hr4, base2d, s02d, s12d, dl2d):
    HALF = N // 2

    @functools.partial(
        pl.kernel,
        out_type=jax.ShapeDtypeStruct((4, N, 128), _f32),
        mesh=_MESH,
        compiler_params=pltpu.CompilerParams(needs_layout_passes=False),
        scratch_types=[pltpu.VMEM((NCH, CHK), _i32),
                       pltpu.VMEM((NCH, CHK), _f32),
                       pltpu.VMEM((NCH, CHK), _i32),
                       pltpu.VMEM((CHK, 128), _f32),
                       pltpu.VMEM((CHK,), _i32),
                       pltpu.VMEM((8, 128), _f32),
                       pltpu.VMEM_SHARED((HALF, 128), _f32),
                       pltpu.SemaphoreType.DMA],
    )
    def k(hr_h, base_h, s0_h, s1_h, dl_h, out_h, bv, sv, dv, rows, gidx,
          zb, accq, sem):
        c = lax.axis_index("c")
        s = lax.axis_index("s")
        row0 = s * NCH
        pltpu.sync_copy(base_h.at[pl.ds(row0, NCH)], bv)
        pltpu.sync_copy(dl_h.at[pl.ds(row0, NCH)], dv)

        @pl.when(c == 0)
        def _():
            pltpu.sync_copy(s0_h.at[pl.ds(row0, NCH)], sv)

        @pl.when(c == 1)
        def _():
            pltpu.sync_copy(s1_h.at[pl.ds(row0, NCH)], sv)

        zv = jnp.zeros((16,), _f32)

        def zb_body(i, _):
            for kk in range(8):
                zb[i, pl.ds(kk * 16, 16)] = zv
            return ()

        lax.fori_loop(0, 8, zb_body, ())
        start = s * 312
        hoff = c * HALF

        for q in range(4):
            def zrow(z, _):
                pltpu.sync_copy(zb, accq.at[pl.ds(start + z * 8, 8)])
                return ()

            lax.fori_loop(0, 39, zrow, ())

            @pl.when(s == 15)
            def _():
                pltpu.sync_copy(zb, accq.at[pl.ds(4992, 8)])

            plsc.subcore_barrier()

            def chunk(j, _):
                def gi(i, _):
                    sl = pl.ds(i * 16, 16)
                    gidx[sl] = bv[j, sl] * 4 + q
                    return ()

                lax.fori_loop(0, CHK // 16, gi, (), unroll=8)
                pltpu.async_copy(hr_h.at[gidx], rows, sem).wait()

                def scale(g2, _):
                    e0 = g2 * 16
                    sgrp = sv[j, pl.ds(e0, 16)]
                    for ee in range(16):
                        se = sgrp[ee]
                        for kk in range(8):
                            sl = pl.ds(kk * 16, 16)
                            rows[e0 + ee, sl] = se * rows[e0 + ee, sl]
                    return ()

                lax.fori_loop(0, CHK // 16, scale, ())
                pltpu.sync_copy(rows, accq.at[dv.at[j]], add=True)
                return ()

            lax.fori_loop(0, NCH, chunk, ())
            plsc.subcore_barrier()

            @pl.when(s < 15)
            def _():
                pltpu.sync_copy(accq.at[pl.ds(start, 312)],
                                out_h.at[q, pl.ds(hoff + start, 312)])

            @pl.when(s == 15)
            def _():
                pltpu.sync_copy(accq.at[pl.ds(4680, 320)],
                                out_h.at[q, pl.ds(hoff + 4680, 320)])

            plsc.subcore_barrier()

    return k(hr4, base2d, s02d, s12d, dl2d)


def kernel(x, edge_index, edge_type, batch, type_, emb0, emb1, emb2, emb3,
           emb4, emb5, W1, root1, b1, W2, root2, b2, gate_l1_w, gate_l1_b,
           bn_gamma, bn_beta, gate_l2_w, gate_l2_b, graph_w, graph_b):
    tables = [emb0, emb1, emb2, emb3, emb4, emb5]
    sizes = [t.shape[0] for t in tables]
    offs = [0]
    for v in sizes:
        offs.append(offs[-1] + v)

    bmat = jnp.zeros((128, DIN), _f32)
    for i, t in enumerate(tables):
        bmat = bmat.at[offs[i]:offs[i + 1], i * IN:(i + 1) * IN].set(t)

    xoff = x.astype(_i32) + jnp.asarray(offs[:6], _i32)[None, :]
    xoff = jnp.concatenate(
        [xoff, jnp.full((N, 2), 127, _i32)], axis=1)

    w1cat = W1.transpose(1, 0, 2).reshape(DIN, DCAT)
    w2cat = W2.transpose(1, 0, 2).reshape(LAYER, DCAT)

    src = edge_index[0].astype(_i32)
    dst = edge_index[1].astype(_i32)
    et = edge_type.astype(_i32)

    wx, rx = _tc_prep(bmat, w1cat, root1)
    hr1, pre1 = _tc_layer1(xoff, wx, rx, b1.reshape(1, LAYER))

    parts = _sc_counts(dst, et)
    base, s0v, s1v, dlv = _sc_edgeprep(parts, dst, et, src)

    pad = EP - E
    base2d = jnp.pad(base, (0, pad)).reshape(EP // CHK, CHK)
    s02d = jnp.pad(s0v, (0, pad)).reshape(EP // CHK, CHK)
    s12d = jnp.pad(s1v, (0, pad)).reshape(EP // CHK, CHK)
    dl2d = jnp.pad(dlv, (0, pad)).reshape(EP // CHK, CHK)

    acc1 = _sc_aggregate(hr1.reshape(N * 16, 128), base2d, s02d, s12d, dl2d)
    hr2, pre2 = _tc_layer2(pre1, acc1, w2cat, root2, b2.reshape(1, LAYER))
    acc2 = _sc_aggregate(hr2.reshape(N * 16, 128), base2d, s02d, s12d, dl2d)

    mu, g, sums = _tc_post(pre2, acc2, gate_l1_w, gate_l1_b.reshape(1, OUT))
    bcol = batch.astype(_f32).reshape(N, 1)
    gate, gmax = _tc_gate(g, bcol, sums, bn_gamma.reshape(1, OUT),
                          bn_beta.reshape(1, OUT), gate_l2_w.reshape(1, OUT))
    return _tc_final(gate, bcol, gmax, mu, graph_w, graph_b.reshape(1, 1))


# speedup vs baseline: 8.0856x; 2.7336x over previous
import functools

import jax
import jax.numpy as jnp
from jax import lax
from jax.experimental import pallas as pl
from jax.experimental.pallas import tpu as pltpu
from jax.experimental.pallas import tpu_sc as plsc

N = 10000
E = 320000
R = 4
IN = 128
LAYER = 512
OUT = 256
NG = 16
DIN = 6 * IN
DCAT = R * LAYER
NR = N * R
NRP = 40960
NB = 1000
GRID = N // NB
EPT = E // 32
CHK = 128
EP = 327680
NCH = EP // 16 // CHK
NROW = N // 16
GS = 640
BCH = 32

_f32 = jnp.float32
_i32 = jnp.int32


def _prep_body(b_ref, w1_ref, r1_ref, wx_ref, rx_ref):
    b = b_ref[...]
    wx_ref[...] = jnp.dot(b, w1_ref[...], preferred_element_type=_f32)
    rx_ref[...] = jnp.dot(b, r1_ref[...], preferred_element_type=_f32)


def _tc_prep(bmat, w1cat, root1):
    return pl.pallas_call(
        _prep_body,
        out_shape=(jax.ShapeDtypeStruct((128, DCAT), _f32),
                   jax.ShapeDtypeStruct((128, LAYER), _f32)),
    )(bmat, w1cat, root1)


def _l1_body(xo_ref, wx_ref, rx_ref, b1_ref, hr_ref, pre_ref):
    xo = xo_ref[...]
    col = lax.broadcasted_iota(_i32, (NB, 128), 1)
    oh = jnp.zeros((NB, 128), _f32)
    for i in range(8):
        oh = oh + (col == xo[:, i:i + 1]).astype(_f32)
    hr_ref[...] = jnp.dot(oh, wx_ref[...], preferred_element_type=_f32)
    pre_ref[...] = (jnp.dot(oh, rx_ref[...], preferred_element_type=_f32)
                    + b1_ref[...])


def _tc_layer1(xoff, wx, rx, b1):
    return pl.pallas_call(
        _l1_body,
        grid=(GRID,),
        in_specs=[
            pl.BlockSpec((NB, 8), lambda i: (i, 0)),
            pl.BlockSpec((128, DCAT), lambda i: (0, 0)),
            pl.BlockSpec((128, LAYER), lambda i: (0, 0)),
            pl.BlockSpec((1, LAYER), lambda i: (0, 0)),
        ],
        out_specs=(pl.BlockSpec((NB, DCAT), lambda i: (i, 0)),
                   pl.BlockSpec((NB, LAYER), lambda i: (i, 0))),
        out_shape=(jax.ShapeDtypeStruct((N, DCAT), _f32),
                   jax.ShapeDtypeStruct((N, LAYER), _f32)),
    )(xoff, wx, rx, b1)


def _l2_body(pre_ref, acc_ref, w2_ref, r2_ref, b2_ref, hr_ref, pre2_ref):
    h = jax.nn.sigmoid(pre_ref[...] + acc_ref[...])
    hr_ref[...] = jnp.dot(h, w2_ref[...], preferred_element_type=_f32)
    pre2_ref[...] = (jnp.dot(h, r2_ref[...], preferred_element_type=_f32)
                     + b2_ref[...])


def _tc_layer2(pre1, acc1, w2cat, root2, b2):
    return pl.pallas_call(
        _l2_body,
        grid=(GRID,),
        in_specs=[
            pl.BlockSpec((NB, LAYER), lambda i: (i, 0)),
            pl.BlockSpec((NB, 512), lambda i: (i, 0)),
            pl.BlockSpec((LAYER, DCAT), lambda i: (0, 0)),
            pl.BlockSpec((LAYER, LAYER), lambda i: (0, 0)),
            pl.BlockSpec((1, LAYER), lambda i: (0, 0)),
        ],
        out_specs=(pl.BlockSpec((NB, DCAT), lambda i: (i, 0)),
                   pl.BlockSpec((NB, LAYER), lambda i: (i, 0))),
        out_shape=(jax.ShapeDtypeStruct((N, DCAT), _f32),
                   jax.ShapeDtypeStruct((N, LAYER), _f32)),
    )(pre1, acc1, w2cat, root2, b2)


def _post_body(pre_ref, acc_ref, gw_ref, gb_ref, mu_ref, g_ref, sums_ref,
               s_acc):
    i = pl.program_id(0)
    h = jax.nn.sigmoid(pre_ref[...] + acc_ref[...])
    mu = h[:, :OUT]
    g = jnp.dot(mu, gw_ref[...], preferred_element_type=_f32) + gb_ref[...]
    mu_ref[...] = mu
    g_ref[...] = g

    @pl.when(i == 0)
    def _():
        s_acc[...] = jnp.zeros_like(s_acc)

    part = jnp.concatenate([jnp.sum(g, axis=0, keepdims=True),
                            jnp.sum(g * g, axis=0, keepdims=True)], axis=0)
    s_acc[...] += part

    @pl.when(i == GRID - 1)
    def _():
        sums_ref[...] = s_acc[...]


def _tc_post(pre2, acc2, gate_l1_w, gate_l1_b):
    return pl.pallas_call(
        _post_body,
        grid=(GRID,),
        in_specs=[
            pl.BlockSpec((NB, LAYER), lambda i: (i, 0)),
            pl.BlockSpec((NB, 512), lambda i: (i, 0)),
            pl.BlockSpec((OUT, OUT), lambda i: (0, 0)),
            pl.BlockSpec((1, OUT), lambda i: (0, 0)),
        ],
        out_specs=(pl.BlockSpec((NB, OUT), lambda i: (i, 0)),
                   pl.BlockSpec((NB, OUT), lambda i: (i, 0)),
                   pl.BlockSpec((2, OUT), lambda i: (0, 0))),
        out_shape=(jax.ShapeDtypeStruct((N, OUT), _f32),
                   jax.ShapeDtypeStruct((N, OUT), _f32),
                   jax.ShapeDtypeStruct((2, OUT), _f32)),
        scratch_shapes=[pltpu.VMEM((2, OUT), _f32)],
    )(pre2, acc2, gate_l1_w, gate_l1_b)


def _gate_body(g_ref, b_ref, sums_ref, gam_ref, bet_ref, w2_ref, gate_ref,
               gmax_ref, m_acc):
    i = pl.program_id(0)
    sums = sums_ref[...]
    mean = sums[0:1, :] / N
    var = sums[1:2, :] / N - mean * mean
    gn = (g_ref[...] - mean) * lax.rsqrt(var + 1e-5) * gam_ref[...] + bet_ref[...]
    gn = jnp.maximum(gn, 0.0)
    gate = jnp.sum(gn * w2_ref[...], axis=1, keepdims=True)
    gate_ref[...] = gate
    seg = lax.broadcasted_iota(_i32, (NB, NG), 1).astype(_f32)
    m = (b_ref[...] == seg)
    masked = jnp.where(m, jnp.broadcast_to(gate, (NB, NG)), -1e30)

    @pl.when(i == 0)
    def _():
        m_acc[...] = jnp.full_like(m_acc, -1e30)

    m_acc[...] = jnp.maximum(m_acc[...], jnp.max(masked, axis=0,
                                                 keepdims=True))

    @pl.when(i == GRID - 1)
    def _():
        gmax_ref[...] = m_acc[...]


def _tc_gate(g, bcol, sums, bn_gamma, bn_beta, w2row):
    return pl.pallas_call(
        _gate_body,
        grid=(GRID,),
        in_specs=[
            pl.BlockSpec((NB, OUT), lambda i: (i, 0)),
            pl.BlockSpec((NB, 1), lambda i: (i, 0)),
            pl.BlockSpec((2, OUT), lambda i: (0, 0)),
            pl.BlockSpec((1, OUT), lambda i: (0, 0)),
            pl.BlockSpec((1, OUT), lambda i: (0, 0)),
            pl.BlockSpec((1, OUT), lambda i: (0, 0)),
        ],
        out_specs=(pl.BlockSpec((NB, 1), lambda i: (i, 0)),
                   pl.BlockSpec((1, NG), lambda i: (0, 0))),
        out_shape=(jax.ShapeDtypeStruct((N, 1), _f32),
                   jax.ShapeDtypeStruct((1, NG), _f32)),
        scratch_shapes=[pltpu.VMEM((1, NG), _f32)],
    )(g, bcol, sums, bn_gamma, bn_beta, w2row)


def _final_body(gate_ref, b_ref, gmax_ref, mu_ref, gw_ref, gb_ref, out_ref,
                u_acc, d_acc):
    i = pl.program_id(0)

    @pl.when(i == 0)
    def _():
        u_acc[...] = jnp.zeros_like(u_acc)
        d_acc[...] = jnp.zeros_like(d_acc)

    seg = lax.broadcasted_iota(_i32, (NB, NG), 1).astype(_f32)
    m = (b_ref[...] == seg).astype(_f32)
    gmb = jnp.sum(m * gmax_ref[...], axis=1, keepdims=True)
    ex = jnp.exp(gate_ref[...] - gmb)
    dn = (((0,), (0,)), ((), ()))
    d_acc[...] += lax.dot_general(m, ex, dn, preferred_element_type=_f32)
    u_acc[...] += lax.dot_general(m, ex * mu_ref[...], dn,
                                  preferred_element_type=_f32)

    @pl.when(i == GRID - 1)
    def _():
        pooled = u_acc[...] / jnp.maximum(d_acc[...], 1e-30)
        val = jnp.dot(pooled, gw_ref[...], preferred_element_type=_f32)
        out_ref[...] = jax.nn.sigmoid(val + gb_ref[...])


def _tc_final(gate, bcol, gmax, mu, graph_w, graph_b):
    return pl.pallas_call(
        _final_body,
        grid=(GRID,),
        in_specs=[
            pl.BlockSpec((NB, 1), lambda i: (i, 0)),
            pl.BlockSpec((NB, 1), lambda i: (i, 0)),
            pl.BlockSpec((1, NG), lambda i: (0, 0)),
            pl.BlockSpec((NB, OUT), lambda i: (i, 0)),
            pl.BlockSpec((OUT, 1), lambda i: (0, 0)),
            pl.BlockSpec((1, 1), lambda i: (0, 0)),
        ],
        out_specs=pl.BlockSpec((NG, 1), lambda i: (0, 0)),
        out_shape=jax.ShapeDtypeStruct((NG, 1), _f32),
        scratch_shapes=[pltpu.VMEM((NG, OUT), _f32),
                        pltpu.VMEM((NG, 1), _f32)],
    )(gate, bcol, gmax, mu, graph_w, graph_b)


_MESH = plsc.VectorSubcoreMesh(core_axis_name="c", subcore_axis_name="s")


def _sc_counts(dst, et):
    @functools.partial(
        pl.kernel,
        out_type=jax.ShapeDtypeStruct((32, NRP), _f32),
        mesh=_MESH,
        compiler_params=pltpu.CompilerParams(needs_layout_passes=False),
        scratch_types=[pltpu.VMEM((EPT,), _i32),
                       pltpu.VMEM((EPT,), _i32),
                       pltpu.VMEM((NRP,), _f32)],
    )
    def k(dst_h, et_h, out_h, dv, tv, cnt):
        c = lax.axis_index("c")
        s = lax.axis_index("s")
        wid = s * 2 + c
        base = wid * EPT
        pltpu.sync_copy(dst_h.at[pl.ds(base, EPT)], dv)
        pltpu.sync_copy(et_h.at[pl.ds(base, EPT)], tv)
        zv = jnp.zeros((16,), _f32)

        def zbody(i, _):
            cnt[pl.ds(i * 16, 16)] = zv
            return ()

        lax.fori_loop(0, NRP // 16, zbody, (), unroll=8)
        ones = jnp.ones((16,), _f32)

        def body(i, _):
            d = dv[pl.ds(i * 16, 16)]
            t = tv[pl.ds(i * 16, 16)]
            plsc.addupdate_scatter(cnt, [d * R + t], ones)
            return ()

        lax.fori_loop(0, EPT // 16, body, (), unroll=4)
        pltpu.sync_copy(cnt, out_h.at[wid])

    return k(dst, et)


def _sc_edgeprep(parts, dst, et, src):
    @functools.partial(
        pl.kernel,
        out_type=(jax.ShapeDtypeStruct((16 * 32 * 10240,), _i32),
                  jax.ShapeDtypeStruct((16 * 32 * 10240,), _f32),
                  jax.ShapeDtypeStruct((16 * 32 * 10240,), _i32),
                  jax.ShapeDtypeStruct((32, 16), _i32)),
        mesh=_MESH,
        compiler_params=pltpu.CompilerParams(needs_layout_passes=False),
        scratch_types=[pltpu.VMEM((NRP,), _f32),
                       pltpu.VMEM((1280,), _f32),
                       pltpu.VMEM((1280,), _f32),
                       pltpu.VMEM((EPT,), _i32),
                       pltpu.VMEM((EPT,), _i32),
                       pltpu.VMEM((EPT,), _i32),
                       pltpu.VMEM((10240,), _i32),
                       pltpu.VMEM((10240,), _f32),
                       pltpu.VMEM((10240,), _i32),
                       pltpu.VMEM((16,), _i32),
                       pltpu.VMEM_SHARED((NRP,), _f32)],
    )
    def k(parts_h, dst_h, et_h, src_h, b_h, s_h, d_h, chunks_h,
          recip_l, cbuf, pbuf, dv, tv, sv, ob, os, od, cnt16, recip_sh):
        c = lax.axis_index("c")
        s = lax.axis_index("s")
        wid = s * 2 + c
        zv = jnp.zeros((16,), _f32)
        for rep in range(2):
            cid = rep * 16 + s
            off = cid * 1280

            def zbody(i, _):
                cbuf[pl.ds(i * 16, 16)] = zv
                return ()

            lax.fori_loop(0, 80, zbody, (), unroll=8)

            def pad(p, _):
                pltpu.sync_copy(parts_h.at[p, pl.ds(off, 1280)], pbuf)

                def abody(i, _):
                    sl = pl.ds(i * 16, 16)
                    cbuf[sl] = cbuf[sl] + pbuf[sl]
                    return ()

                lax.fori_loop(0, 80, abody, (), unroll=8)
                return ()

            lax.fori_loop(0, 32, pad, ())

            def rbody(i, _):
                sl = pl.ds(i * 16, 16)
                cbuf[sl] = 1.0 / jnp.maximum(cbuf[sl], 1.0)
                return ()

            lax.fori_loop(0, 80, rbody, (), unroll=8)
            pltpu.sync_copy(cbuf, recip_sh.at[pl.ds(off, 1280)])
        plsc.subcore_barrier()
        pltpu.sync_copy(recip_sh, recip_l)
        base = wid * EPT
        pltpu.sync_copy(dst_h.at[pl.ds(base, EPT)], dv)
        pltpu.sync_copy(et_h.at[pl.ds(base, EPT)], tv)
        pltpu.sync_copy(src_h.at[pl.ds(base, EPT)], sv)
        ilane = lax.iota(_i32, 16)
        cnt16[...] = jnp.zeros((16,), _i32)
        zvi = jnp.zeros((16,), _i32)

        def clr0(i, _):
            sl = pl.ds(i * 16, 16)
            ob[sl] = zvi
            od[sl] = zvi
            return ()

        lax.fori_loop(0, 640, clr0, (), unroll=8)

        for g in range(16):
            def clr(i, _):
                os[pl.ds(i * 16, 16)] = zv
                return ()

            lax.fori_loop(0, 640, clr, (), unroll=8)

            def body(i, off, g=g):
                sl = pl.ds(i * 16, 16)
                d = dv[sl]
                t = tv[sl]
                r = plsc.load_gather(recip_l, [d * R + t])
                if g == 0:
                    m = d < GS
                elif g == 15:
                    m = d >= 15 * GS
                else:
                    m = jnp.logical_and(d >= g * GS, d < (g + 1) * GS)
                dl = d - g * GS
                bi = sv[sl] * R + t
                plsc.store_compressed(ob.at[pl.ds(off, 16)], bi, mask=m)
                plsc.store_compressed(os.at[pl.ds(off, 16)], r, mask=m)
                plsc.store_compressed(od.at[pl.ds(off, 16)], dl, mask=m)
                return off + plsc.all_reduce_population_count(m)[0]

            n = lax.fori_loop(0, EPT // 16, body, jnp.int32(0))
            nch = (n + (BCH - 1)) >> 5
            cnt16[...] = cnt16[...] + jnp.where(ilane == g, nch, 0)
            obase = (g * 32 + wid) * 10240
            pltpu.sync_copy(ob, b_h.at[pl.ds(obase, 10240)])
            pltpu.sync_copy(os, s_h.at[pl.ds(obase, 10240)])
            pltpu.sync_copy(od, d_h.at[pl.ds(obase, 10240)])
        pltpu.sync_copy(cnt16, chunks_h.at[wid])

    return k(parts, dst, et, src)


def _sc_aggregate(hr4, ball, sall, dall, chunks):
    @functools.partial(
        pl.kernel,
        out_type=jax.ShapeDtypeStruct((N * 4, 128), _f32),
        mesh=_MESH,
        compiler_params=pltpu.CompilerParams(needs_layout_passes=False),
        scratch_types=[pltpu.VMEM((10240,), _i32),
                       pltpu.VMEM((10240,), _f32),
                       pltpu.VMEM((10240,), _i32),
                       pltpu.VMEM((BCH, 512), _f32),
                       pltpu.VMEM((BCH, 512), _f32),
                       pltpu.VMEM((4 * BCH, 128), _f32),
                       pltpu.VMEM((4 * BCH, 128), _f32),
                       pltpu.VMEM((16,), _i32),
                       pltpu.VMEM((16,), _i32),
                       pltpu.VMEM((4 * BCH,), _i32),
                       pltpu.VMEM((4 * BCH,), _i32),
                       pltpu.VMEM((8, 128), _f32),
                       pltpu.VMEM_SHARED((4 * GS, 128), _f32),
                       pltpu.SemaphoreType.DMA,
                       pltpu.SemaphoreType.DMA,
                       pltpu.SemaphoreType.DMA,
                       pltpu.SemaphoreType.DMA],
    )
    def k(hr_h, b_h, s_h, d_h, chunks_h, out_h,
          bv, sv, dv, rows0, rows1, sc0, sc1, cr0, cr1, gix0, gix1, zb,
          accq, sem0, sem1, sems0, sems1):
        c = lax.axis_index("c")
        s = lax.axis_index("s")
        pltpu.sync_copy(chunks_h.at[2 * s], cr0)
        pltpu.sync_copy(chunks_h.at[2 * s + 1], cr1)
        zv = jnp.zeros((16,), _f32)

        def zb_body(i, _):
            for kk in range(8):
                zb[i, pl.ds(kk * 16, 16)] = zv
            return ()

        lax.fori_loop(0, 8, zb_body, ())
        start = s * 160

        lane16 = lax.iota(_i32, 16)

        def process(rows, scr, j, sem, gix):
            def scale(g2, _):
                e0 = g2 * 16
                sgrp = sv[pl.ds(j * BCH + e0, 16)]
                for ee in range(16):
                    se = sgrp[ee]
                    for kk in range(32):
                        sl = pl.ds((kk % 8) * 16, 16)
                        scr[(e0 + ee) * 4 + kk // 8, sl] = (
                            se * rows[e0 + ee, pl.ds(kk * 16, 16)])
                return ()

            lax.fori_loop(0, BCH // 16, scale, ())

            def gi(i, _):
                lane = lane16 + i * 16
                dvals = plsc.load_gather(dv, [j * BCH + (lane >> 2)])
                gix[pl.ds(i * 16, 16)] = (dvals * 4
                                          + jnp.bitwise_and(lane, 3))
                return ()

            lax.fori_loop(0, 4 * BCH // 16, gi, (), unroll=2)
            pltpu.async_copy(scr, accq.at[gix], sem, add=True)

        def drain_scatter(scr, sem, gix):
            pltpu.make_async_copy(scr, accq.at[gix], sem).wait()

        def gbody(gl, _):
            g = c * 8 + gl
            na = plsc.load_gather(cr0, [jnp.full((16,), g, _i32)])[0]
            nb = plsc.load_gather(cr1, [jnp.full((16,), g, _i32)])[0]

            def zrow(z, _):
                pltpu.sync_copy(zb, accq.at[pl.ds(start + z * 8, 8)])
                return ()

            lax.fori_loop(0, 20, zrow, ())
            plsc.subcore_barrier()

            def regloop(reg, _):
                rbase = (g * 32 + 2 * s + reg) * 10240
                pltpu.sync_copy(b_h.at[pl.ds(rbase, 10240)], bv)
                pltpu.sync_copy(s_h.at[pl.ds(rbase, 10240)], sv)
                pltpu.sync_copy(d_h.at[pl.ds(rbase, 10240)], dv)
                ntot = jnp.where(reg == 0, na, nb)

                @pl.when(ntot > 0)
                def _():
                    pltpu.async_copy(hr_h.at[bv.at[pl.ds(0, BCH)]],
                                     rows0, sem0)

                def body(j, _):
                    even = (j % 2) == 0

                    def step(rA, scA, sA, ssA, giA, rB, scB, sB, ssB, giB):
                        @pl.when(j > 0)
                        def _():
                            drain_scatter(scB, ssB, giB)

                        @pl.when(j + 1 < ntot)
                        def _():
                            pltpu.async_copy(
                                hr_h.at[bv.at[pl.ds((j + 1) * BCH, BCH)]],
                                rB, sB)

                        pltpu.make_async_copy(
                            hr_h.at[bv.at[pl.ds(j * BCH, BCH)]],
                            rA, sA).wait()
                        process(rA, scA, j, ssA, giA)

                    @pl.when(even)
                    def _():
                        step(rows0, sc0, sem0, sems0, gix0,
                             rows1, sc1, sem1, sems1, gix1)

                    @pl.when(jnp.logical_not(even))
                    def _():
                        step(rows1, sc1, sem1, sems1, gix1,
                             rows0, sc0, sem0, sems0, gix0)

                    return ()

                lax.fori_loop(0, ntot, body, ())

                @pl.when(jnp.logical_and(ntot > 0, (ntot - 1) % 2 == 0))
                def _():
                    drain_scatter(sc0, sems0, gix0)

                @pl.when(jnp.logical_and(ntot > 0, (ntot - 1) % 2 == 1))
                def _():
                    drain_scatter(sc1, sems1, gix1)

                return ()

            lax.fori_loop(0, 2, regloop, ())
            plsc.subcore_barrier()
            goff = g * GS * 4
            @pl.when(jnp.logical_or(g < 15, s < 10))
            def _():
                pltpu.sync_copy(accq.at[pl.ds(start, 160)],
                                out_h.at[pl.ds(goff + start, 160)])

            plsc.subcore_barrier()
            return ()

        lax.fori_loop(0, 8, gbody, ())

    return k(hr4, ball, sall, dall, chunks)


def kernel(x, edge_index, edge_type, batch, type_, emb0, emb1, emb2, emb3,
           emb4, emb5, W1, root1, b1, W2, root2, b2, gate_l1_w, gate_l1_b,
           bn_gamma, bn_beta, gate_l2_w, gate_l2_b, graph_w, graph_b):
    tables = [emb0, emb1, emb2, emb3, emb4, emb5]
    sizes = [t.shape[0] for t in tables]
    offs = [0]
    for v in sizes:
        offs.append(offs[-1] + v)

    bmat = jnp.zeros((128, DIN), _f32)
    for i, t in enumerate(tables):
        bmat = bmat.at[offs[i]:offs[i + 1], i * IN:(i + 1) * IN].set(t)

    xoff = x.astype(_i32) + jnp.asarray(offs[:6], _i32)[None, :]
    xoff = jnp.concatenate(
        [xoff, jnp.full((N, 2), 127, _i32)], axis=1)

    w1cat = W1.transpose(1, 0, 2).reshape(DIN, DCAT)
    w2cat = W2.transpose(1, 0, 2).reshape(LAYER, DCAT)

    src = edge_index[0].astype(_i32)
    dst = edge_index[1].astype(_i32)
    et = edge_type.astype(_i32)

    wx, rx = _tc_prep(bmat, w1cat, root1)
    hr1, pre1 = _tc_layer1(xoff, wx, rx, b1.reshape(1, LAYER))

    parts = _sc_counts(dst, et)
    ball, sall, dall, chunks = _sc_edgeprep(parts, dst, et, src)

    acc1 = _sc_aggregate(hr1.reshape(N * 4, 512), ball, sall, dall,
                         chunks).reshape(N, 512)
    hr2, pre2 = _tc_layer2(pre1, acc1, w2cat, root2, b2.reshape(1, LAYER))
    acc2 = _sc_aggregate(hr2.reshape(N * 4, 512), ball, sall, dall,
                         chunks).reshape(N, 512)

    mu, g, sums = _tc_post(pre2, acc2, gate_l1_w, gate_l1_b.reshape(1, OUT))
    bcol = batch.astype(_f32).reshape(N, 1)
    gate, gmax = _tc_gate(g, bcol, sums, bn_gamma.reshape(1, OUT),
                          bn_beta.reshape(1, OUT), gate_l2_w.reshape(1, OUT))
    return _tc_final(gate, bcol, gmax, mu, graph_w, graph_b.reshape(1, 1))


# speedup vs baseline: 8.0897x; 1.0005x over previous
import functools

import jax
import jax.numpy as jnp
from jax import lax
from jax.experimental import pallas as pl
from jax.experimental.pallas import tpu as pltpu
from jax.experimental.pallas import tpu_sc as plsc

N = 10000
E = 320000
R = 4
IN = 128
LAYER = 512
OUT = 256
NG = 16
DIN = 6 * IN
DCAT = R * LAYER
NR = N * R
NRP = 40960
NB = 1000
GRID = N // NB
EPT = E // 32
CHK = 128
EP = 327680
NCH = EP // 16 // CHK
NROW = N // 16
GS = 640
BCH = 32

_f32 = jnp.float32
_i32 = jnp.int32


def _prep_body(b_ref, w1_ref, r1_ref, wx_ref, rx_ref):
    b = b_ref[...]
    wx_ref[...] = jnp.dot(b, w1_ref[...], preferred_element_type=_f32)
    rx_ref[...] = jnp.dot(b, r1_ref[...], preferred_element_type=_f32)


def _tc_prep(bmat, w1cat, root1):
    return pl.pallas_call(
        _prep_body,
        out_shape=(jax.ShapeDtypeStruct((128, DCAT), _f32),
                   jax.ShapeDtypeStruct((128, LAYER), _f32)),
    )(bmat, w1cat, root1)


def _l1_body(xo_ref, wx_ref, rx_ref, b1_ref, hr_ref, pre_ref):
    xo = xo_ref[...]
    col = lax.broadcasted_iota(_i32, (NB, 128), 1)
    oh = jnp.zeros((NB, 128), _f32)
    for i in range(8):
        oh = oh + (col == xo[:, i:i + 1]).astype(_f32)
    hr_ref[...] = jnp.dot(oh, wx_ref[...], preferred_element_type=_f32)
    pre_ref[...] = (jnp.dot(oh, rx_ref[...], preferred_element_type=_f32)
                    + b1_ref[...])


def _tc_layer1(xoff, wx, rx, b1):
    return pl.pallas_call(
        _l1_body,
        grid=(GRID,),
        in_specs=[
            pl.BlockSpec((NB, 8), lambda i: (i, 0)),
            pl.BlockSpec((128, DCAT), lambda i: (0, 0)),
            pl.BlockSpec((128, LAYER), lambda i: (0, 0)),
            pl.BlockSpec((1, LAYER), lambda i: (0, 0)),
        ],
        out_specs=(pl.BlockSpec((NB, DCAT), lambda i: (i, 0)),
                   pl.BlockSpec((NB, LAYER), lambda i: (i, 0))),
        out_shape=(jax.ShapeDtypeStruct((N, DCAT), _f32),
                   jax.ShapeDtypeStruct((N, LAYER), _f32)),
    )(xoff, wx, rx, b1)


def _l2_body(pre_ref, acc_ref, w2_ref, r2_ref, b2_ref, hr_ref, pre2_ref):
    h = jax.nn.sigmoid(pre_ref[...] + acc_ref[...])
    hr_ref[...] = jnp.dot(h, w2_ref[...], preferred_element_type=_f32)
    pre2_ref[...] = (jnp.dot(h, r2_ref[...], preferred_element_type=_f32)
                     + b2_ref[...])


def _tc_layer2(pre1, acc1, w2cat, root2, b2):
    return pl.pallas_call(
        _l2_body,
        grid=(GRID,),
        in_specs=[
            pl.BlockSpec((NB, LAYER), lambda i: (i, 0)),
            pl.BlockSpec((NB, 512), lambda i: (i, 0)),
            pl.BlockSpec((LAYER, DCAT), lambda i: (0, 0)),
            pl.BlockSpec((LAYER, LAYER), lambda i: (0, 0)),
            pl.BlockSpec((1, LAYER), lambda i: (0, 0)),
        ],
        out_specs=(pl.BlockSpec((NB, DCAT), lambda i: (i, 0)),
                   pl.BlockSpec((NB, LAYER), lambda i: (i, 0))),
        out_shape=(jax.ShapeDtypeStruct((N, DCAT), _f32),
                   jax.ShapeDtypeStruct((N, LAYER), _f32)),
    )(pre1, acc1, w2cat, root2, b2)


def _post_body(pre_ref, acc_ref, gw_ref, gb_ref, mu_ref, g_ref, sums_ref,
               s_acc):
    i = pl.program_id(0)
    h = jax.nn.sigmoid(pre_ref[...] + acc_ref[...])
    mu = h[:, :OUT]
    g = jnp.dot(mu, gw_ref[...], preferred_element_type=_f32) + gb_ref[...]
    mu_ref[...] = mu
    g_ref[...] = g

    @pl.when(i == 0)
    def _():
        s_acc[...] = jnp.zeros_like(s_acc)

    part = jnp.concatenate([jnp.sum(g, axis=0, keepdims=True),
                            jnp.sum(g * g, axis=0, keepdims=True)], axis=0)
    s_acc[...] += part

    @pl.when(i == GRID - 1)
    def _():
        sums_ref[...] = s_acc[...]


def _tc_post(pre2, acc2, gate_l1_w, gate_l1_b):
    return pl.pallas_call(
        _post_body,
        grid=(GRID,),
        in_specs=[
            pl.BlockSpec((NB, LAYER), lambda i: (i, 0)),
            pl.BlockSpec((NB, 512), lambda i: (i, 0)),
            pl.BlockSpec((OUT, OUT), lambda i: (0, 0)),
            pl.BlockSpec((1, OUT), lambda i: (0, 0)),
        ],
        out_specs=(pl.BlockSpec((NB, OUT), lambda i: (i, 0)),
                   pl.BlockSpec((NB, OUT), lambda i: (i, 0)),
                   pl.BlockSpec((2, OUT), lambda i: (0, 0))),
        out_shape=(jax.ShapeDtypeStruct((N, OUT), _f32),
                   jax.ShapeDtypeStruct((N, OUT), _f32),
                   jax.ShapeDtypeStruct((2, OUT), _f32)),
        scratch_shapes=[pltpu.VMEM((2, OUT), _f32)],
    )(pre2, acc2, gate_l1_w, gate_l1_b)


def _gate_body(g_ref, b_ref, sums_ref, gam_ref, bet_ref, w2_ref, gate_ref,
               gmax_ref, m_acc):
    i = pl.program_id(0)
    sums = sums_ref[...]
    mean = sums[0:1, :] / N
    var = sums[1:2, :] / N - mean * mean
    gn = (g_ref[...] - mean) * lax.rsqrt(var + 1e-5) * gam_ref[...] + bet_ref[...]
    gn = jnp.maximum(gn, 0.0)
    gate = jnp.sum(gn * w2_ref[...], axis=1, keepdims=True)
    gate_ref[...] = gate
    seg = lax.broadcasted_iota(_i32, (NB, NG), 1).astype(_f32)
    m = (b_ref[...] == seg)
    masked = jnp.where(m, jnp.broadcast_to(gate, (NB, NG)), -1e30)

    @pl.when(i == 0)
    def _():
        m_acc[...] = jnp.full_like(m_acc, -1e30)

    m_acc[...] = jnp.maximum(m_acc[...], jnp.max(masked, axis=0,
                                                 keepdims=True))

    @pl.when(i == GRID - 1)
    def _():
        gmax_ref[...] = m_acc[...]


def _tc_gate(g, bcol, sums, bn_gamma, bn_beta, w2row):
    return pl.pallas_call(
        _gate_body,
        grid=(GRID,),
        in_specs=[
            pl.BlockSpec((NB, OUT), lambda i: (i, 0)),
            pl.BlockSpec((NB, 1), lambda i: (i, 0)),
            pl.BlockSpec((2, OUT), lambda i: (0, 0)),
            pl.BlockSpec((1, OUT), lambda i: (0, 0)),
            pl.BlockSpec((1, OUT), lambda i: (0, 0)),
            pl.BlockSpec((1, OUT), lambda i: (0, 0)),
        ],
        out_specs=(pl.BlockSpec((NB, 1), lambda i: (i, 0)),
                   pl.BlockSpec((1, NG), lambda i: (0, 0))),
        out_shape=(jax.ShapeDtypeStruct((N, 1), _f32),
                   jax.ShapeDtypeStruct((1, NG), _f32)),
        scratch_shapes=[pltpu.VMEM((1, NG), _f32)],
    )(g, bcol, sums, bn_gamma, bn_beta, w2row)


def _final_body(gate_ref, b_ref, gmax_ref, mu_ref, gw_ref, gb_ref, out_ref,
                u_acc, d_acc):
    i = pl.program_id(0)

    @pl.when(i == 0)
    def _():
        u_acc[...] = jnp.zeros_like(u_acc)
        d_acc[...] = jnp.zeros_like(d_acc)

    seg = lax.broadcasted_iota(_i32, (NB, NG), 1).astype(_f32)
    m = (b_ref[...] == seg).astype(_f32)
    gmb = jnp.sum(m * gmax_ref[...], axis=1, keepdims=True)
    ex = jnp.exp(gate_ref[...] - gmb)
    dn = (((0,), (0,)), ((), ()))
    d_acc[...] += lax.dot_general(m, ex, dn, preferred_element_type=_f32)
    u_acc[...] += lax.dot_general(m, ex * mu_ref[...], dn,
                                  preferred_element_type=_f32)

    @pl.when(i == GRID - 1)
    def _():
        pooled = u_acc[...] / jnp.maximum(d_acc[...], 1e-30)
        val = jnp.dot(pooled, gw_ref[...], preferred_element_type=_f32)
        out_ref[...] = jax.nn.sigmoid(val + gb_ref[...])


def _tc_final(gate, bcol, gmax, mu, graph_w, graph_b):
    return pl.pallas_call(
        _final_body,
        grid=(GRID,),
        in_specs=[
            pl.BlockSpec((NB, 1), lambda i: (i, 0)),
            pl.BlockSpec((NB, 1), lambda i: (i, 0)),
            pl.BlockSpec((1, NG), lambda i: (0, 0)),
            pl.BlockSpec((NB, OUT), lambda i: (i, 0)),
            pl.BlockSpec((OUT, 1), lambda i: (0, 0)),
            pl.BlockSpec((1, 1), lambda i: (0, 0)),
        ],
        out_specs=pl.BlockSpec((NG, 1), lambda i: (0, 0)),
        out_shape=jax.ShapeDtypeStruct((NG, 1), _f32),
        scratch_shapes=[pltpu.VMEM((NG, OUT), _f32),
                        pltpu.VMEM((NG, 1), _f32)],
    )(gate, bcol, gmax, mu, graph_w, graph_b)


_MESH = plsc.VectorSubcoreMesh(core_axis_name="c", subcore_axis_name="s")


def _sc_counts(dst, et):
    @functools.partial(
        pl.kernel,
        out_type=jax.ShapeDtypeStruct((32, NRP), _f32),
        mesh=_MESH,
        compiler_params=pltpu.CompilerParams(needs_layout_passes=False),
        scratch_types=[pltpu.VMEM((EPT,), _i32),
                       pltpu.VMEM((EPT,), _i32),
                       pltpu.VMEM((NRP,), _f32)],
    )
    def k(dst_h, et_h, out_h, dv, tv, cnt):
        c = lax.axis_index("c")
        s = lax.axis_index("s")
        wid = s * 2 + c
        base = wid * EPT
        pltpu.sync_copy(dst_h.at[pl.ds(base, EPT)], dv)
        pltpu.sync_copy(et_h.at[pl.ds(base, EPT)], tv)
        zv = jnp.zeros((16,), _f32)

        def zbody(i, _):
            cnt[pl.ds(i * 16, 16)] = zv
            return ()

        lax.fori_loop(0, NRP // 16, zbody, (), unroll=8)
        ones = jnp.ones((16,), _f32)

        def body(i, _):
            d = dv[pl.ds(i * 16, 16)]
            t = tv[pl.ds(i * 16, 16)]
            plsc.addupdate_scatter(cnt, [d * R + t], ones)
            return ()

        lax.fori_loop(0, EPT // 16, body, (), unroll=4)
        pltpu.sync_copy(cnt, out_h.at[wid])

    return k(dst, et)


def _sc_edgeprep(parts, dst, et, src):
    @functools.partial(
        pl.kernel,
        out_type=(jax.ShapeDtypeStruct((16 * 32 * 10240,), _i32),
                  jax.ShapeDtypeStruct((16 * 32 * 10240,), _f32),
                  jax.ShapeDtypeStruct((16 * 32 * 10240,), _i32),
                  jax.ShapeDtypeStruct((32, 16), _i32)),
        mesh=_MESH,
        compiler_params=pltpu.CompilerParams(needs_layout_passes=False),
        scratch_types=[pltpu.VMEM((NRP,), _f32),
                       pltpu.VMEM((1280,), _f32),
                       pltpu.VMEM((1280,), _f32),
                       pltpu.VMEM((EPT,), _i32),
                       pltpu.VMEM((EPT,), _i32),
                       pltpu.VMEM((EPT,), _i32),
                       pltpu.VMEM((10240,), _i32),
                       pltpu.VMEM((10240,), _f32),
                       pltpu.VMEM((10240,), _i32),
                       pltpu.VMEM((16,), _i32),
                       pltpu.VMEM_SHARED((NRP,), _f32)],
    )
    def k(parts_h, dst_h, et_h, src_h, b_h, s_h, d_h, chunks_h,
          recip_l, cbuf, pbuf, dv, tv, sv, ob, os, od, cnt16, recip_sh):
        c = lax.axis_index("c")
        s = lax.axis_index("s")
        wid = s * 2 + c
        zv = jnp.zeros((16,), _f32)
        for rep in range(2):
            cid = rep * 16 + s
            off = cid * 1280

            def zbody(i, _):
                cbuf[pl.ds(i * 16, 16)] = zv
                return ()

            lax.fori_loop(0, 80, zbody, (), unroll=8)

            def pad(p, _):
                pltpu.sync_copy(parts_h.at[p, pl.ds(off, 1280)], pbuf)

                def abody(i, _):
                    sl = pl.ds(i * 16, 16)
                    cbuf[sl] = cbuf[sl] + pbuf[sl]
                    return ()

                lax.fori_loop(0, 80, abody, (), unroll=8)
                return ()

            lax.fori_loop(0, 32, pad, ())

            def rbody(i, _):
                sl = pl.ds(i * 16, 16)
                cbuf[sl] = 1.0 / jnp.maximum(cbuf[sl], 1.0)
                return ()

            lax.fori_loop(0, 80, rbody, (), unroll=8)
            pltpu.sync_copy(cbuf, recip_sh.at[pl.ds(off, 1280)])
        plsc.subcore_barrier()
        pltpu.sync_copy(recip_sh, recip_l)
        base = wid * EPT
        pltpu.sync_copy(dst_h.at[pl.ds(base, EPT)], dv)
        pltpu.sync_copy(et_h.at[pl.ds(base, EPT)], tv)
        pltpu.sync_copy(src_h.at[pl.ds(base, EPT)], sv)
        ilane = lax.iota(_i32, 16)
        cnt16[...] = jnp.zeros((16,), _i32)
        zvi = jnp.zeros((16,), _i32)

        def prep(i, _):
            sl = pl.ds(i * 16, 16)
            d = dv[sl]
            t = tv[sl]
            rv = plsc.load_gather(recip_l, [d * R + t])
            tv[sl] = plsc.bitcast(rv, _i32)
            sv[sl] = sv[sl] * R + t
            return ()

        lax.fori_loop(0, EPT // 16, prep, (), unroll=4)

        def clr0(i, _):
            sl = pl.ds(i * 16, 16)
            ob[sl] = zvi
            od[sl] = zvi
            return ()

        lax.fori_loop(0, 640, clr0, (), unroll=8)

        for g in range(16):
            def clr(i, _):
                os[pl.ds(i * 16, 16)] = zv
                return ()

            lax.fori_loop(0, 640, clr, (), unroll=8)

            def body(i, off, g=g):
                sl = pl.ds(i * 16, 16)
                d = dv[sl]
                if g == 0:
                    m = d < GS
                elif g == 15:
                    m = d >= 15 * GS
                else:
                    m = jnp.logical_and(d >= g * GS, d < (g + 1) * GS)
                plsc.store_compressed(ob.at[pl.ds(off, 16)], sv[sl], mask=m)
                plsc.store_compressed(os.at[pl.ds(off, 16)],
                                      plsc.bitcast(tv[sl], _f32), mask=m)
                plsc.store_compressed(od.at[pl.ds(off, 16)], d - g * GS,
                                      mask=m)
                return off + plsc.all_reduce_population_count(m)[0]

            n = lax.fori_loop(0, EPT // 16, body, jnp.int32(0))
            nch = (n + (BCH - 1)) >> 5
            cnt16[...] = cnt16[...] + jnp.where(ilane == g, nch, 0)
            obase = (g * 32 + wid) * 10240
            pltpu.sync_copy(ob, b_h.at[pl.ds(obase, 10240)])
            pltpu.sync_copy(os, s_h.at[pl.ds(obase, 10240)])
            pltpu.sync_copy(od, d_h.at[pl.ds(obase, 10240)])
        pltpu.sync_copy(cnt16, chunks_h.at[wid])

    return k(parts, dst, et, src)


def _sc_aggregate(hr4, ball, sall, dall, chunks):
    @functools.partial(
        pl.kernel,
        out_type=jax.ShapeDtypeStruct((N * 4, 128), _f32),
        mesh=_MESH,
        compiler_params=pltpu.CompilerParams(needs_layout_passes=False),
        scratch_types=[pltpu.VMEM((10240,), _i32),
                       pltpu.VMEM((10240,), _f32),
                       pltpu.VMEM((10240,), _i32),
                       pltpu.VMEM((BCH, 512), _f32),
                       pltpu.VMEM((BCH, 512), _f32),
                       pltpu.VMEM((4 * BCH, 128), _f32),
                       pltpu.VMEM((4 * BCH, 128), _f32),
                       pltpu.VMEM((16,), _i32),
                       pltpu.VMEM((16,), _i32),
                       pltpu.VMEM((4 * BCH,), _i32),
                       pltpu.VMEM((4 * BCH,), _i32),
                       pltpu.VMEM((8, 128), _f32),
                       pltpu.VMEM_SHARED((4 * GS, 128), _f32),
                       pltpu.SemaphoreType.DMA,
                       pltpu.SemaphoreType.DMA,
                       pltpu.SemaphoreType.DMA,
                       pltpu.SemaphoreType.DMA],
    )
    def k(hr_h, b_h, s_h, d_h, chunks_h, out_h,
          bv, sv, dv, rows0, rows1, sc0, sc1, cr0, cr1, gix0, gix1, zb,
          accq, sem0, sem1, sems0, sems1):
        c = lax.axis_index("c")
        s = lax.axis_index("s")
        pltpu.sync_copy(chunks_h.at[2 * s], cr0)
        pltpu.sync_copy(chunks_h.at[2 * s + 1], cr1)
        zv = jnp.zeros((16,), _f32)

        def zb_body(i, _):
            for kk in range(8):
                zb[i, pl.ds(kk * 16, 16)] = zv
            return ()

        lax.fori_loop(0, 8, zb_body, ())
        start = s * 160

        lane16 = lax.iota(_i32, 16)

        def process(rows, scr, j, sem, gix):
            def scale(g2, _):
                e0 = g2 * 16
                sgrp = sv[pl.ds(j * BCH + e0, 16)]
                for ee in range(16):
                    se = sgrp[ee]
                    for kk in range(32):
                        sl = pl.ds((kk % 8) * 16, 16)
                        scr[(e0 + ee) * 4 + kk // 8, sl] = (
                            se * rows[e0 + ee, pl.ds(kk * 16, 16)])
                return ()

            lax.fori_loop(0, BCH // 16, scale, ())

            def gi(i, _):
                lane = lane16 + i * 16
                dvals = plsc.load_gather(dv, [j * BCH + (lane >> 2)])
                gix[pl.ds(i * 16, 16)] = (dvals * 4
                                          + jnp.bitwise_and(lane, 3))
                return ()

            lax.fori_loop(0, 4 * BCH // 16, gi, (), unroll=2)
            pltpu.async_copy(scr, accq.at[gix], sem, add=True)

        def drain_scatter(scr, sem, gix):
            pltpu.make_async_copy(scr, accq.at[gix], sem).wait()

        def gbody(gl, _):
            g = c * 8 + gl
            na = plsc.load_gather(cr0, [jnp.full((16,), g, _i32)])[0]
            nb = plsc.load_gather(cr1, [jnp.full((16,), g, _i32)])[0]

            def zrow(z, _):
                pltpu.sync_copy(zb, accq.at[pl.ds(start + z * 8, 8)])
                return ()

            lax.fori_loop(0, 20, zrow, ())
            plsc.subcore_barrier()

            def regloop(reg, _):
                rbase = (g * 32 + 2 * s + reg) * 10240
                pltpu.sync_copy(b_h.at[pl.ds(rbase, 10240)], bv)
                pltpu.sync_copy(s_h.at[pl.ds(rbase, 10240)], sv)
                pltpu.sync_copy(d_h.at[pl.ds(rbase, 10240)], dv)
                ntot = jnp.where(reg == 0, na, nb)

                @pl.when(ntot > 0)
                def _():
                    pltpu.async_copy(hr_h.at[bv.at[pl.ds(0, BCH)]],
                                     rows0, sem0)

                def body(j, _):
                    even = (j % 2) == 0

                    def step(rA, scA, sA, ssA, giA, rB, scB, sB, ssB, giB):
                        @pl.when(j > 0)
                        def _():
                            drain_scatter(scB, ssB, giB)

                        @pl.when(j + 1 < ntot)
                        def _():
                            pltpu.async_copy(
                                hr_h.at[bv.at[pl.ds((j + 1) * BCH, BCH)]],
                                rB, sB)

                        pltpu.make_async_copy(
                            hr_h.at[bv.at[pl.ds(j * BCH, BCH)]],
                            rA, sA).wait()
                        process(rA, scA, j, ssA, giA)

                    @pl.when(even)
                    def _():
                        step(rows0, sc0, sem0, sems0, gix0,
                             rows1, sc1, sem1, sems1, gix1)

                    @pl.when(jnp.logical_not(even))
                    def _():
                        step(rows1, sc1, sem1, sems1, gix1,
                             rows0, sc0, sem0, sems0, gix0)

                    return ()

                lax.fori_loop(0, ntot, body, ())

                @pl.when(jnp.logical_and(ntot > 0, (ntot - 1) % 2 == 0))
                def _():
                    drain_scatter(sc0, sems0, gix0)

                @pl.when(jnp.logical_and(ntot > 0, (ntot - 1) % 2 == 1))
                def _():
                    drain_scatter(sc1, sems1, gix1)

                return ()

            lax.fori_loop(0, 2, regloop, ())
            plsc.subcore_barrier()
            goff = g * GS * 4
            @pl.when(jnp.logical_or(g < 15, s < 10))
            def _():
                pltpu.sync_copy(accq.at[pl.ds(start, 160)],
                                out_h.at[pl.ds(goff + start, 160)])

            plsc.subcore_barrier()
            return ()

        lax.fori_loop(0, 8, gbody, ())

    return k(hr4, ball, sall, dall, chunks)


def kernel(x, edge_index, edge_type, batch, type_, emb0, emb1, emb2, emb3,
           emb4, emb5, W1, root1, b1, W2, root2, b2, gate_l1_w, gate_l1_b,
           bn_gamma, bn_beta, gate_l2_w, gate_l2_b, graph_w, graph_b):
    tables = [emb0, emb1, emb2, emb3, emb4, emb5]
    sizes = [t.shape[0] for t in tables]
    offs = [0]
    for v in sizes:
        offs.append(offs[-1] + v)

    bmat = jnp.zeros((128, DIN), _f32)
    for i, t in enumerate(tables):
        bmat = bmat.at[offs[i]:offs[i + 1], i * IN:(i + 1) * IN].set(t)

    xoff = x.astype(_i32) + jnp.asarray(offs[:6], _i32)[None, :]
    xoff = jnp.concatenate(
        [xoff, jnp.full((N, 2), 127, _i32)], axis=1)

    w1cat = W1.transpose(1, 0, 2).reshape(DIN, DCAT)
    w2cat = W2.transpose(1, 0, 2).reshape(LAYER, DCAT)

    src = edge_index[0].astype(_i32)
    dst = edge_index[1].astype(_i32)
    et = edge_type.astype(_i32)

    wx, rx = _tc_prep(bmat, w1cat, root1)
    hr1, pre1 = _tc_layer1(xoff, wx, rx, b1.reshape(1, LAYER))

    parts = _sc_counts(dst, et)
    ball, sall, dall, chunks = _sc_edgeprep(parts, dst, et, src)

    acc1 = _sc_aggregate(hr1.reshape(N * 4, 512), ball, sall, dall,
                         chunks).reshape(N, 512)
    hr2, pre2 = _tc_layer2(pre1, acc1, w2cat, root2, b2.reshape(1, LAYER))
    acc2 = _sc_aggregate(hr2.reshape(N * 4, 512), ball, sall, dall,
                         chunks).reshape(N, 512)

    mu, g, sums = _tc_post(pre2, acc2, gate_l1_w, gate_l1_b.reshape(1, OUT))
    bcol = batch.astype(_f32).reshape(N, 1)
    gate, gmax = _tc_gate(g, bcol, sums, bn_gamma.reshape(1, OUT),
                          bn_beta.reshape(1, OUT), gate_l2_w.reshape(1, OUT))
    return _tc_final(gate, bcol, gmax, mu, graph_w, graph_b.reshape(1, 1))


# speedup vs baseline: 8.1410x; 1.0063x over previous
import functools

import jax
import jax.numpy as jnp
from jax import lax
from jax.experimental import pallas as pl
from jax.experimental.pallas import tpu as pltpu
from jax.experimental.pallas import tpu_sc as plsc

N = 10000
E = 320000
R = 4
IN = 128
LAYER = 512
OUT = 256
NG = 16
DIN = 6 * IN
DCAT = R * LAYER
NR = N * R
NRP = 40960
NB = 1000
GRID = N // NB
EPT = E // 32
CHK = 128
EP = 327680
NCH = EP // 16 // CHK
NROW = N // 16
GS = 640
BCH = 32

_f32 = jnp.float32
_i32 = jnp.int32


def _prep_body(b_ref, w1_ref, r1_ref, wx_ref, rx_ref):
    b = b_ref[...]
    wx_ref[...] = jnp.dot(b, w1_ref[...], preferred_element_type=_f32)
    rx_ref[...] = jnp.dot(b, r1_ref[...], preferred_element_type=_f32)


def _tc_prep(bmat, w1cat, root1):
    return pl.pallas_call(
        _prep_body,
        out_shape=(jax.ShapeDtypeStruct((128, DCAT), _f32),
                   jax.ShapeDtypeStruct((128, LAYER), _f32)),
    )(bmat, w1cat, root1)


def _l1_body(xo_ref, wx_ref, rx_ref, b1_ref, hr_ref, pre_ref):
    xo = xo_ref[...]
    col = lax.broadcasted_iota(_i32, (NB, 128), 1)
    oh = jnp.zeros((NB, 128), _f32)
    for i in range(8):
        oh = oh + (col == xo[:, i:i + 1]).astype(_f32)
    hr_ref[...] = jnp.dot(oh, wx_ref[...], preferred_element_type=_f32)
    pre_ref[...] = (jnp.dot(oh, rx_ref[...], preferred_element_type=_f32)
                    + b1_ref[...])


def _tc_layer1(xoff, wx, rx, b1):
    return pl.pallas_call(
        _l1_body,
        grid=(GRID,),
        in_specs=[
            pl.BlockSpec((NB, 8), lambda i: (i, 0)),
            pl.BlockSpec((128, DCAT), lambda i: (0, 0)),
            pl.BlockSpec((128, LAYER), lambda i: (0, 0)),
            pl.BlockSpec((1, LAYER), lambda i: (0, 0)),
        ],
        out_specs=(pl.BlockSpec((NB, DCAT), lambda i: (i, 0)),
                   pl.BlockSpec((NB, LAYER), lambda i: (i, 0))),
        out_shape=(jax.ShapeDtypeStruct((N, DCAT), _f32),
                   jax.ShapeDtypeStruct((N, LAYER), _f32)),
    )(xoff, wx, rx, b1)


def _l2_body(pre_ref, acc_ref, w2_ref, r2_ref, b2_ref, hr_ref, pre2_ref):
    h = jax.nn.sigmoid(pre_ref[...] + acc_ref[...])
    hr_ref[...] = jnp.dot(h, w2_ref[...], preferred_element_type=_f32)
    pre2_ref[...] = (jnp.dot(h, r2_ref[...], preferred_element_type=_f32)
                     + b2_ref[...])


def _tc_layer2(pre1, acc1, w2cat, root2, b2):
    return pl.pallas_call(
        _l2_body,
        grid=(GRID,),
        in_specs=[
            pl.BlockSpec((NB, LAYER), lambda i: (i, 0)),
            pl.BlockSpec((NB, 512), lambda i: (i, 0)),
            pl.BlockSpec((LAYER, DCAT), lambda i: (0, 0)),
            pl.BlockSpec((LAYER, LAYER), lambda i: (0, 0)),
            pl.BlockSpec((1, LAYER), lambda i: (0, 0)),
        ],
        out_specs=(pl.BlockSpec((NB, DCAT), lambda i: (i, 0)),
                   pl.BlockSpec((NB, LAYER), lambda i: (i, 0))),
        out_shape=(jax.ShapeDtypeStruct((N, DCAT), _f32),
                   jax.ShapeDtypeStruct((N, LAYER), _f32)),
    )(pre1, acc1, w2cat, root2, b2)


def _post_body(pre_ref, acc_ref, gw_ref, gb_ref, mu_ref, g_ref, sums_ref,
               s_acc):
    i = pl.program_id(0)
    h = jax.nn.sigmoid(pre_ref[...] + acc_ref[...])
    mu = h[:, :OUT]
    g = jnp.dot(mu, gw_ref[...], preferred_element_type=_f32) + gb_ref[...]
    mu_ref[...] = mu
    g_ref[...] = g

    @pl.when(i == 0)
    def _():
        s_acc[...] = jnp.zeros_like(s_acc)

    part = jnp.concatenate([jnp.sum(g, axis=0, keepdims=True),
                            jnp.sum(g * g, axis=0, keepdims=True)], axis=0)
    s_acc[...] += part

    @pl.when(i == GRID - 1)
    def _():
        sums_ref[...] = s_acc[...]


def _tc_post(pre2, acc2, gate_l1_w, gate_l1_b):
    return pl.pallas_call(
        _post_body,
        grid=(GRID,),
        in_specs=[
            pl.BlockSpec((NB, LAYER), lambda i: (i, 0)),
            pl.BlockSpec((NB, 512), lambda i: (i, 0)),
            pl.BlockSpec((OUT, OUT), lambda i: (0, 0)),
            pl.BlockSpec((1, OUT), lambda i: (0, 0)),
        ],
        out_specs=(pl.BlockSpec((NB, OUT), lambda i: (i, 0)),
                   pl.BlockSpec((NB, OUT), lambda i: (i, 0)),
                   pl.BlockSpec((2, OUT), lambda i: (0, 0))),
        out_shape=(jax.ShapeDtypeStruct((N, OUT), _f32),
                   jax.ShapeDtypeStruct((N, OUT), _f32),
                   jax.ShapeDtypeStruct((2, OUT), _f32)),
        scratch_shapes=[pltpu.VMEM((2, OUT), _f32)],
    )(pre2, acc2, gate_l1_w, gate_l1_b)


def _gate_body(g_ref, b_ref, sums_ref, gam_ref, bet_ref, w2_ref, gate_ref,
               gmax_ref, m_acc):
    i = pl.program_id(0)
    sums = sums_ref[...]
    mean = sums[0:1, :] / N
    var = sums[1:2, :] / N - mean * mean
    gn = (g_ref[...] - mean) * lax.rsqrt(var + 1e-5) * gam_ref[...] + bet_ref[...]
    gn = jnp.maximum(gn, 0.0)
    gate = jnp.sum(gn * w2_ref[...], axis=1, keepdims=True)
    gate_ref[...] = gate
    seg = lax.broadcasted_iota(_i32, (NB, NG), 1).astype(_f32)
    m = (b_ref[...] == seg)
    masked = jnp.where(m, jnp.broadcast_to(gate, (NB, NG)), -1e30)

    @pl.when(i == 0)
    def _():
        m_acc[...] = jnp.full_like(m_acc, -1e30)

    m_acc[...] = jnp.maximum(m_acc[...], jnp.max(masked, axis=0,
                                                 keepdims=True))

    @pl.when(i == GRID - 1)
    def _():
        gmax_ref[...] = m_acc[...]


def _tc_gate(g, bcol, sums, bn_gamma, bn_beta, w2row):
    return pl.pallas_call(
        _gate_body,
        grid=(GRID,),
        in_specs=[
            pl.BlockSpec((NB, OUT), lambda i: (i, 0)),
            pl.BlockSpec((NB, 1), lambda i: (i, 0)),
            pl.BlockSpec((2, OUT), lambda i: (0, 0)),
            pl.BlockSpec((1, OUT), lambda i: (0, 0)),
            pl.BlockSpec((1, OUT), lambda i: (0, 0)),
            pl.BlockSpec((1, OUT), lambda i: (0, 0)),
        ],
        out_specs=(pl.BlockSpec((NB, 1), lambda i: (i, 0)),
                   pl.BlockSpec((1, NG), lambda i: (0, 0))),
        out_shape=(jax.ShapeDtypeStruct((N, 1), _f32),
                   jax.ShapeDtypeStruct((1, NG), _f32)),
        scratch_shapes=[pltpu.VMEM((1, NG), _f32)],
    )(g, bcol, sums, bn_gamma, bn_beta, w2row)


def _final_body(gate_ref, b_ref, gmax_ref, mu_ref, gw_ref, gb_ref, out_ref,
                u_acc, d_acc):
    i = pl.program_id(0)

    @pl.when(i == 0)
    def _():
        u_acc[...] = jnp.zeros_like(u_acc)
        d_acc[...] = jnp.zeros_like(d_acc)

    seg = lax.broadcasted_iota(_i32, (NB, NG), 1).astype(_f32)
    m = (b_ref[...] == seg).astype(_f32)
    gmb = jnp.sum(m * gmax_ref[...], axis=1, keepdims=True)
    ex = jnp.exp(gate_ref[...] - gmb)
    dn = (((0,), (0,)), ((), ()))
    d_acc[...] += lax.dot_general(m, ex, dn, preferred_element_type=_f32)
    u_acc[...] += lax.dot_general(m, ex * mu_ref[...], dn,
                                  preferred_element_type=_f32)

    @pl.when(i == GRID - 1)
    def _():
        pooled = u_acc[...] / jnp.maximum(d_acc[...], 1e-30)
        val = jnp.dot(pooled, gw_ref[...], preferred_element_type=_f32)
        out_ref[...] = jax.nn.sigmoid(val + gb_ref[...])


def _tc_final(gate, bcol, gmax, mu, graph_w, graph_b):
    return pl.pallas_call(
        _final_body,
        grid=(GRID,),
        in_specs=[
            pl.BlockSpec((NB, 1), lambda i: (i, 0)),
            pl.BlockSpec((NB, 1), lambda i: (i, 0)),
            pl.BlockSpec((1, NG), lambda i: (0, 0)),
            pl.BlockSpec((NB, OUT), lambda i: (i, 0)),
            pl.BlockSpec((OUT, 1), lambda i: (0, 0)),
            pl.BlockSpec((1, 1), lambda i: (0, 0)),
        ],
        out_specs=pl.BlockSpec((NG, 1), lambda i: (0, 0)),
        out_shape=jax.ShapeDtypeStruct((NG, 1), _f32),
        scratch_shapes=[pltpu.VMEM((NG, OUT), _f32),
                        pltpu.VMEM((NG, 1), _f32)],
    )(gate, bcol, gmax, mu, graph_w, graph_b)


_MESH = plsc.VectorSubcoreMesh(core_axis_name="c", subcore_axis_name="s")


def _sc_counts(dst, et):
    @functools.partial(
        pl.kernel,
        out_type=jax.ShapeDtypeStruct((32, NRP), _f32),
        mesh=_MESH,
        compiler_params=pltpu.CompilerParams(needs_layout_passes=False),
        scratch_types=[pltpu.VMEM((EPT,), _i32),
                       pltpu.VMEM((EPT,), _i32),
                       pltpu.VMEM((NRP,), _f32)],
    )
    def k(dst_h, et_h, out_h, dv, tv, cnt):
        c = lax.axis_index("c")
        s = lax.axis_index("s")
        wid = s * 2 + c
        base = wid * EPT
        pltpu.sync_copy(dst_h.at[pl.ds(base, EPT)], dv)
        pltpu.sync_copy(et_h.at[pl.ds(base, EPT)], tv)
        zv = jnp.zeros((16,), _f32)

        def zbody(i, _):
            cnt[pl.ds(i * 16, 16)] = zv
            return ()

        lax.fori_loop(0, NRP // 16, zbody, (), unroll=8)
        ones = jnp.ones((16,), _f32)

        def body(i, _):
            d = dv[pl.ds(i * 16, 16)]
            t = tv[pl.ds(i * 16, 16)]
            plsc.addupdate_scatter(cnt, [d * R + t], ones)
            return ()

        lax.fori_loop(0, EPT // 16, body, (), unroll=4)
        pltpu.sync_copy(cnt, out_h.at[wid])

    return k(dst, et)


def _sc_edgeprep(parts, dst, et, src):
    @functools.partial(
        pl.kernel,
        out_type=(jax.ShapeDtypeStruct((16 * 32 * 10240,), _i32),
                  jax.ShapeDtypeStruct((16 * 32 * 10240,), _f32),
                  jax.ShapeDtypeStruct((16 * 32 * 10240,), _i32),
                  jax.ShapeDtypeStruct((32, 16), _i32)),
        mesh=_MESH,
        compiler_params=pltpu.CompilerParams(needs_layout_passes=False),
        scratch_types=[pltpu.VMEM((NRP,), _f32),
                       pltpu.VMEM((1280,), _f32),
                       pltpu.VMEM((1280,), _f32),
                       pltpu.VMEM((EPT,), _i32),
                       pltpu.VMEM((EPT,), _i32),
                       pltpu.VMEM((EPT,), _i32),
                       pltpu.VMEM((10240,), _i32),
                       pltpu.VMEM((10240,), _f32),
                       pltpu.VMEM((10240,), _i32),
                       pltpu.VMEM((16,), _i32),
                       pltpu.VMEM_SHARED((NRP,), _f32)],
    )
    def k(parts_h, dst_h, et_h, src_h, b_h, s_h, d_h, chunks_h,
          recip_l, cbuf, pbuf, dv, tv, sv, ob, os, od, cnt16, recip_sh):
        c = lax.axis_index("c")
        s = lax.axis_index("s")
        wid = s * 2 + c
        zv = jnp.zeros((16,), _f32)
        for rep in range(2):
            cid = rep * 16 + s
            off = cid * 1280

            def zbody(i, _):
                cbuf[pl.ds(i * 16, 16)] = zv
                return ()

            lax.fori_loop(0, 80, zbody, (), unroll=8)

            def pad(p, _):
                pltpu.sync_copy(parts_h.at[p, pl.ds(off, 1280)], pbuf)

                def abody(i, _):
                    sl = pl.ds(i * 16, 16)
                    cbuf[sl] = cbuf[sl] + pbuf[sl]
                    return ()

                lax.fori_loop(0, 80, abody, (), unroll=8)
                return ()

            lax.fori_loop(0, 32, pad, ())

            def rbody(i, _):
                sl = pl.ds(i * 16, 16)
                cbuf[sl] = 1.0 / jnp.maximum(cbuf[sl], 1.0)
                return ()

            lax.fori_loop(0, 80, rbody, (), unroll=8)
            pltpu.sync_copy(cbuf, recip_sh.at[pl.ds(off, 1280)])
        plsc.subcore_barrier()
        pltpu.sync_copy(recip_sh, recip_l)
        base = wid * EPT
        pltpu.sync_copy(dst_h.at[pl.ds(base, EPT)], dv)
        pltpu.sync_copy(et_h.at[pl.ds(base, EPT)], tv)
        pltpu.sync_copy(src_h.at[pl.ds(base, EPT)], sv)
        ilane = lax.iota(_i32, 16)
        cnt16[...] = jnp.zeros((16,), _i32)
        zvi = jnp.zeros((16,), _i32)

        def prep(i, _):
            sl = pl.ds(i * 16, 16)
            d = dv[sl]
            t = tv[sl]
            rv = plsc.load_gather(recip_l, [d * R + t])
            tv[sl] = plsc.bitcast(rv, _i32)
            sv[sl] = sv[sl] * R + t
            return ()

        lax.fori_loop(0, EPT // 16, prep, (), unroll=4)

        def clr0(i, _):
            sl = pl.ds(i * 16, 16)
            ob[sl] = zvi
            od[sl] = zvi
            return ()

        lax.fori_loop(0, 640, clr0, (), unroll=8)

        for g in range(16):
            def clr(i, _):
                os[pl.ds(i * 16, 16)] = zv
                return ()

            lax.fori_loop(0, 640, clr, (), unroll=8)

            def body(i, off, g=g):
                sl = pl.ds(i * 16, 16)
                d = dv[sl]
                if g == 0:
                    m = d < GS
                elif g == 15:
                    m = d >= 15 * GS
                else:
                    m = jnp.logical_and(d >= g * GS, d < (g + 1) * GS)
                plsc.store_compressed(ob.at[pl.ds(off, 16)], sv[sl], mask=m)
                plsc.store_compressed(os.at[pl.ds(off, 16)],
                                      plsc.bitcast(tv[sl], _f32), mask=m)
                plsc.store_compressed(od.at[pl.ds(off, 16)], d - g * GS,
                                      mask=m)
                return off + plsc.all_reduce_population_count(m)[0]

            n = lax.fori_loop(0, EPT // 16, body, jnp.int32(0))
            nch = (n + (BCH - 1)) >> 5
            cnt16[...] = cnt16[...] + jnp.where(ilane == g, nch, 0)
            obase = (g * 32 + wid) * 10240
            pltpu.sync_copy(ob, b_h.at[pl.ds(obase, 10240)])
            pltpu.sync_copy(os, s_h.at[pl.ds(obase, 10240)])
            pltpu.sync_copy(od, d_h.at[pl.ds(obase, 10240)])
        pltpu.sync_copy(cnt16, chunks_h.at[wid])

    return k(parts, dst, et, src)


def _sc_aggregate(hr4, ball, sall, dall, chunks):
    @functools.partial(
        pl.kernel,
        out_type=jax.ShapeDtypeStruct((N * 4, 128), _f32),
        mesh=_MESH,
        compiler_params=pltpu.CompilerParams(needs_layout_passes=False),
        scratch_types=[pltpu.VMEM((10240,), _i32),
                       pltpu.VMEM((10240,), _f32),
                       pltpu.VMEM((10240,), _i32),
                       pltpu.VMEM((BCH, 512), _f32),
                       pltpu.VMEM((BCH, 512), _f32),
                       pltpu.VMEM((4 * BCH, 128), _f32),
                       pltpu.VMEM((4 * BCH, 128), _f32),
                       pltpu.VMEM((16,), _i32),
                       pltpu.VMEM((16,), _i32),
                       pltpu.VMEM((4 * BCH,), _i32),
                       pltpu.VMEM((4 * BCH,), _i32),
                       pltpu.VMEM((40, 128), _f32),
                       pltpu.VMEM_SHARED((4 * GS, 128), _f32),
                       pltpu.SemaphoreType.DMA,
                       pltpu.SemaphoreType.DMA,
                       pltpu.SemaphoreType.DMA,
                       pltpu.SemaphoreType.DMA],
    )
    def k(hr_h, b_h, s_h, d_h, chunks_h, out_h,
          bv, sv, dv, rows0, rows1, sc0, sc1, cr0, cr1, gix0, gix1, zb,
          accq, sem0, sem1, sems0, sems1):
        c = lax.axis_index("c")
        s = lax.axis_index("s")
        pltpu.sync_copy(chunks_h.at[2 * s], cr0)
        pltpu.sync_copy(chunks_h.at[2 * s + 1], cr1)
        zv = jnp.zeros((16,), _f32)

        def zb_body(i, _):
            for kk in range(8):
                zb[i, pl.ds(kk * 16, 16)] = zv
            return ()

        lax.fori_loop(0, 40, zb_body, ())
        start = s * 160

        lane16 = lax.iota(_i32, 16)

        def process(rows, scr, j, sem, gix):
            def scale(g2, _):
                e0 = g2 * 16
                sgrp = sv[pl.ds(j * BCH + e0, 16)]
                for ee in range(16):
                    se = sgrp[ee]
                    for kk in range(32):
                        sl = pl.ds((kk % 8) * 16, 16)
                        scr[(e0 + ee) * 4 + kk // 8, sl] = (
                            se * rows[e0 + ee, pl.ds(kk * 16, 16)])
                return ()

            lax.fori_loop(0, BCH // 16, scale, ())

            def gi(i, _):
                lane = lane16 + i * 16
                dvals = plsc.load_gather(dv, [j * BCH + (lane >> 2)])
                gix[pl.ds(i * 16, 16)] = (dvals * 4
                                          + jnp.bitwise_and(lane, 3))
                return ()

            lax.fori_loop(0, 4 * BCH // 16, gi, (), unroll=2)
            pltpu.async_copy(scr, accq.at[gix], sem, add=True)

        def drain_scatter(scr, sem, gix):
            pltpu.make_async_copy(scr, accq.at[gix], sem).wait()

        def gbody(gl, _):
            g = c * 8 + gl
            na = plsc.load_gather(cr0, [jnp.full((16,), g, _i32)])[0]
            nb = plsc.load_gather(cr1, [jnp.full((16,), g, _i32)])[0]

            def zrow(z, _):
                pltpu.sync_copy(zb, accq.at[pl.ds(start + z * 40, 40)])
                return ()

            lax.fori_loop(0, 4, zrow, ())
            plsc.subcore_barrier()

            def regloop(reg, _):
                rbase = (g * 32 + 2 * s + reg) * 10240
                pltpu.sync_copy(b_h.at[pl.ds(rbase, 10240)], bv)
                pltpu.sync_copy(s_h.at[pl.ds(rbase, 10240)], sv)
                pltpu.sync_copy(d_h.at[pl.ds(rbase, 10240)], dv)
                ntot = jnp.where(reg == 0, na, nb)

                @pl.when(ntot > 0)
                def _():
                    pltpu.async_copy(hr_h.at[bv.at[pl.ds(0, BCH)]],
                                     rows0, sem0)

                def body(j, _):
                    even = (j % 2) == 0

                    def step(rA, scA, sA, ssA, giA, rB, scB, sB, ssB, giB):
                        @pl.when(j > 0)
                        def _():
                            drain_scatter(scB, ssB, giB)

                        @pl.when(j + 1 < ntot)
                        def _():
                            pltpu.async_copy(
                                hr_h.at[bv.at[pl.ds((j + 1) * BCH, BCH)]],
                                rB, sB)

                        pltpu.make_async_copy(
                            hr_h.at[bv.at[pl.ds(j * BCH, BCH)]],
                            rA, sA).wait()
                        process(rA, scA, j, ssA, giA)

                    @pl.when(even)
                    def _():
                        step(rows0, sc0, sem0, sems0, gix0,
                             rows1, sc1, sem1, sems1, gix1)

                    @pl.when(jnp.logical_not(even))
                    def _():
                        step(rows1, sc1, sem1, sems1, gix1,
                             rows0, sc0, sem0, sems0, gix0)

                    return ()

                lax.fori_loop(0, ntot, body, ())

                @pl.when(jnp.logical_and(ntot > 0, (ntot - 1) % 2 == 0))
                def _():
                    drain_scatter(sc0, sems0, gix0)

                @pl.when(jnp.logical_and(ntot > 0, (ntot - 1) % 2 == 1))
                def _():
                    drain_scatter(sc1, sems1, gix1)

                return ()

            lax.fori_loop(0, 2, regloop, ())
            plsc.subcore_barrier()
            goff = g * GS * 4
            @pl.when(jnp.logical_or(g < 15, s < 10))
            def _():
                pltpu.sync_copy(accq.at[pl.ds(start, 160)],
                                out_h.at[pl.ds(goff + start, 160)])

            plsc.subcore_barrier()
            return ()

        lax.fori_loop(0, 8, gbody, ())

    return k(hr4, ball, sall, dall, chunks)


def kernel(x, edge_index, edge_type, batch, type_, emb0, emb1, emb2, emb3,
           emb4, emb5, W1, root1, b1, W2, root2, b2, gate_l1_w, gate_l1_b,
           bn_gamma, bn_beta, gate_l2_w, gate_l2_b, graph_w, graph_b):
    tables = [emb0, emb1, emb2, emb3, emb4, emb5]
    sizes = [t.shape[0] for t in tables]
    offs = [0]
    for v in sizes:
        offs.append(offs[-1] + v)

    bmat = jnp.zeros((128, DIN), _f32)
    for i, t in enumerate(tables):
        bmat = bmat.at[offs[i]:offs[i + 1], i * IN:(i + 1) * IN].set(t)

    xoff = x.astype(_i32) + jnp.asarray(offs[:6], _i32)[None, :]
    xoff = jnp.concatenate(
        [xoff, jnp.full((N, 2), 127, _i32)], axis=1)

    w1cat = W1.transpose(1, 0, 2).reshape(DIN, DCAT)
    w2cat = W2.transpose(1, 0, 2).reshape(LAYER, DCAT)

    src = edge_index[0].astype(_i32)
    dst = edge_index[1].astype(_i32)
    et = edge_type.astype(_i32)

    wx, rx = _tc_prep(bmat, w1cat, root1)
    hr1, pre1 = _tc_layer1(xoff, wx, rx, b1.reshape(1, LAYER))

    parts = _sc_counts(dst, et)
    ball, sall, dall, chunks = _sc_edgeprep(parts, dst, et, src)

    acc1 = _sc_aggregate(hr1.reshape(N * 4, 512), ball, sall, dall,
                         chunks).reshape(N, 512)
    hr2, pre2 = _tc_layer2(pre1, acc1, w2cat, root2, b2.reshape(1, LAYER))
    acc2 = _sc_aggregate(hr2.reshape(N * 4, 512), ball, sall, dall,
                         chunks).reshape(N, 512)

    mu, g, sums = _tc_post(pre2, acc2, gate_l1_w, gate_l1_b.reshape(1, OUT))
    bcol = batch.astype(_f32).reshape(N, 1)
    gate, gmax = _tc_gate(g, bcol, sums, bn_gamma.reshape(1, OUT),
                          bn_beta.reshape(1, OUT), gate_l2_w.reshape(1, OUT))
    return _tc_final(gate, bcol, gmax, mu, graph_w, graph_b.reshape(1, 1))
